# Initial kernel scaffold; baseline (speedup 1.0000x reference)
import jax
import jax.numpy as jnp
from jax.experimental import pallas as pl


def kernel(context_idxs, emb_table, W, b):
    raise NotImplementedError("write your pallas kernel here")


# speedup vs baseline: 9.6313x; 9.6313x over previous
import functools

import jax
import jax.numpy as jnp
from jax import lax
from jax.experimental import pallas as pl
from jax.experimental.pallas import tpu as pltpu
from jax.experimental.pallas import tpu_sc as plsc

VOCAB = 100000
D = 64
BATCH = 16384
HIST = 50

NC = 2
NS = 16
NW = NC * NS

PER_W = BATCH * HIST // NW
G = 2
CHUNK = G * HIST
NCH = PER_W // CHUNK

_mesh = plsc.VectorSubcoreMesh(core_axis_name="c", subcore_axis_name="s")


@functools.partial(
    pl.kernel,
    mesh=_mesh,
    out_type=jax.ShapeDtypeStruct((NW, HIST, D), jnp.float32),
    scratch_types=[
        pltpu.VMEM((NCH, CHUNK), jnp.int32),
        pltpu.VMEM((CHUNK, D), jnp.float32),
        pltpu.VMEM((HIST, D), jnp.float32),
        pltpu.SemaphoreType.DMA,
    ],
    compiler_params=pltpu.CompilerParams(use_tc_tiling_on_sc=False),
)
def _sc_partial_sums(idx_hbm, table_hbm, out_hbm, idxv, rows, acc, gsem):
    wid = lax.axis_index("s") * NC + lax.axis_index("c")

    zero = jnp.zeros((16,), jnp.float32)

    def zbody(l, carry):
        for d in range(D // 16):
            acc[l, pl.ds(d * 16, 16)] = zero
        return carry

    lax.fori_loop(0, HIST, zbody, 0)

    pltpu.sync_copy(idx_hbm.at[wid], idxv)

    def chunk_body(c, carry):
        pltpu.async_copy(table_hbm.at[idxv.at[c]], rows, gsem).wait()
        def abody(l, inner):
            for d in range(D // 16):
                sl = pl.ds(d * 16, 16)
                v = rows[l, sl]
                for g in range(1, G):
                    v = v + rows[g * HIST + l, sl]
                plsc.addupdate(acc.at[l, sl], v)
            return inner

        lax.fori_loop(0, HIST, abody, 0)
        return carry

    lax.fori_loop(0, NCH, chunk_body, 0)

    pltpu.sync_copy(acc, out_hbm.at[wid])


VC = 8192


def _mm_body(part_ref, w_ref, b_ref, o_ref):
    mean = jnp.sum(part_ref[...], axis=0) * (1.0 / BATCH)
    o_ref[...] = (
        lax.dot_general(
            mean, w_ref[...], (((1,), (1,)), ((), ())),
            preferred_element_type=jnp.float32,
        )
        + b_ref[...]
    )


_project = pl.pallas_call(
    _mm_body,
    grid=(pl.cdiv(VOCAB, VC),),
    in_specs=[
        pl.BlockSpec((NW, HIST, D), lambda j: (0, 0, 0)),
        pl.BlockSpec((VC, D), lambda j: (j, 0)),
        pl.BlockSpec((1, VC), lambda j: (0, j)),
    ],
    out_specs=pl.BlockSpec((HIST, VC), lambda j: (0, j)),
    out_shape=jax.ShapeDtypeStruct((HIST, VOCAB), jnp.float32),
)


def kernel(context_idxs, emb_table, W, b):
    idx = context_idxs.astype(jnp.int32).reshape(NW, NCH, CHUNK)
    partials = _sc_partial_sums(idx, emb_table)
    return _project(partials, W, b.reshape(1, VOCAB))


# speedup vs baseline: 18.3353x; 1.9037x over previous
import functools

import jax
import jax.numpy as jnp
from jax import lax
from jax.experimental import pallas as pl
from jax.experimental.pallas import tpu as pltpu
from jax.experimental.pallas import tpu_sc as plsc

VOCAB = 100000
D = 64
BATCH = 16384
HIST = 50

NC = 2
NS = 16
NW = NC * NS

PER_W = BATCH * HIST // NW
G = 2
CHUNK = G * HIST
NCH = PER_W // CHUNK
GRP = 4
RPG = GRP * CHUNK
NG = NCH // GRP

_mesh = plsc.VectorSubcoreMesh(core_axis_name="c", subcore_axis_name="s")


@functools.partial(
    pl.kernel,
    mesh=_mesh,
    out_type=jax.ShapeDtypeStruct((NW, HIST, D), jnp.float32),
    scratch_types=[
        pltpu.VMEM((NCH, CHUNK), jnp.int32),
        pltpu.VMEM((RPG, D), jnp.float32),
        pltpu.VMEM((RPG, D), jnp.float32),
        pltpu.VMEM((HIST, D), jnp.float32),
        pltpu.SemaphoreType.DMA,
        pltpu.SemaphoreType.DMA,
    ],
    compiler_params=pltpu.CompilerParams(use_tc_tiling_on_sc=False),
)
def _sc_partial_sums(idx_hbm, table_hbm, out_hbm, idxv, rows0, rows1, acc, sem0, sem1):
    wid = lax.axis_index("s") * NC + lax.axis_index("c")

    zero = jnp.zeros((16,), jnp.float32)

    def zbody(l, carry):
        for d in range(D // 16):
            acc[l, pl.ds(d * 16, 16)] = zero
        return carry

    lax.fori_loop(0, HIST, zbody, 0)

    pltpu.sync_copy(idx_hbm.at[wid], idxv)

    def fire(grp, rows, sem):
        for k in range(GRP):
            pltpu.async_copy(
                table_hbm.at[idxv.at[grp * GRP + k]],
                rows.at[pl.ds(k * CHUNK, CHUNK)],
                sem,
            )

    def drain(rows, sem):
        pltpu.make_async_copy(table_hbm.at[pl.ds(0, RPG)], rows, sem).wait()

    def accumulate(rows):
        def abody(l, inner):
            for d in range(D // 16):
                sl = pl.ds(d * 16, 16)
                v = rows[l, sl]
                for g in range(1, GRP * G):
                    v = v + rows[g * HIST + l, sl]
                plsc.addupdate(acc.at[l, sl], v)
            return inner

        lax.fori_loop(0, HIST, abody, 0)

    fire(0, rows0, sem0)

    def group_body(i, carry):
        fire(2 * i + 1, rows1, sem1)
        drain(rows0, sem0)
        accumulate(rows0)

        @pl.when(2 * i + 2 < NG)
        def _():
            fire(2 * i + 2, rows0, sem0)

        drain(rows1, sem1)
        accumulate(rows1)
        return carry

    lax.fori_loop(0, NG // 2, group_body, 0)

    pltpu.sync_copy(acc, out_hbm.at[wid])


VC = 8192


def _mm_body(part_ref, w_ref, b_ref, o_ref):
    mean = jnp.sum(part_ref[...], axis=0) * (1.0 / BATCH)
    o_ref[...] = (
        lax.dot_general(
            mean, w_ref[...], (((1,), (1,)), ((), ())),
            preferred_element_type=jnp.float32,
        )
        + b_ref[...]
    )


_project = pl.pallas_call(
    _mm_body,
    grid=(pl.cdiv(VOCAB, VC),),
    in_specs=[
        pl.BlockSpec((NW, HIST, D), lambda j: (0, 0, 0)),
        pl.BlockSpec((VC, D), lambda j: (j, 0)),
        pl.BlockSpec((1, VC), lambda j: (0, j)),
    ],
    out_specs=pl.BlockSpec((HIST, VC), lambda j: (0, j)),
    out_shape=jax.ShapeDtypeStruct((HIST, VOCAB), jnp.float32),
)


def kernel(context_idxs, emb_table, W, b):
    idx = context_idxs.astype(jnp.int32).reshape(NW, NCH, CHUNK)
    partials = _sc_partial_sums(idx, emb_table)
    return _project(partials, W, b.reshape(1, VOCAB))
